# baseline (speedup 1.0000x reference)
import functools

import jax
import jax.numpy as jnp
from jax import lax
from jax.experimental import pallas as pl
from jax.experimental.pallas import tpu as pltpu
from jax.experimental.pallas import tpu_sc as plsc

N_PLAYERS = 1000000
N_COURSES = 100000
EMBED_DIM = 16
BATCH = 16384

NC = 2
NS = 16
NW = NC * NS
BPW = BATCH // NW
CHUNK = 128
NCHUNK = BPW // CHUNK
GROUPS = BPW // 16

_mesh = plsc.VectorSubcoreMesh(core_axis_name="c", subcore_axis_name="s")


@functools.partial(
    pl.kernel,
    out_type=jax.ShapeDtypeStruct((BATCH,), jnp.float32),
    mesh=_mesh,
    compiler_params=pltpu.CompilerParams(
        needs_layout_passes=False, use_tc_tiling_on_sc=False),
    scratch_types=[
        pltpu.VMEM((NCHUNK, CHUNK), jnp.int32),
        pltpu.VMEM((NCHUNK, CHUNK), jnp.int32),
        pltpu.VMEM((BPW, EMBED_DIM), jnp.float32),
        pltpu.VMEM((BPW, EMBED_DIM), jnp.float32),
        pltpu.VMEM((BPW,), jnp.float32),
        pltpu.VMEM((BPW,), jnp.float32),
        pltpu.VMEM((16,), jnp.float32),
        pltpu.VMEM((BPW,), jnp.float32),
        pltpu.SemaphoreType.DMA,
    ],
)
def _sc_kernel(pid_hbm, cid_hbm, pemb_hbm, cemb_hbm, pbias_hbm, cbias_hbm,
               gbias_hbm, out_hbm,
               idx_p, idx_c, prow, crow, pbv, cbv, gbv, outv, sem):
    wid = lax.axis_index("s") * NC + lax.axis_index("c")

    pltpu.sync_copy(pid_hbm.at[wid], idx_p)
    pltpu.sync_copy(cid_hbm.at[wid], idx_c)
    pltpu.sync_copy(gbias_hbm, gbv)

    copies = []
    for j in range(NCHUNK):
        sl = pl.ds(j * CHUNK, CHUNK)
        copies.append(pltpu.async_copy(pemb_hbm.at[idx_p.at[j]], prow.at[sl, :], sem))
        copies.append(pltpu.async_copy(cemb_hbm.at[idx_c.at[j]], crow.at[sl, :], sem))
        copies.append(pltpu.async_copy(pbias_hbm.at[idx_p.at[j]], pbv.at[sl], sem))
        copies.append(pltpu.async_copy(cbias_hbm.at[idx_c.at[j]], cbv.at[sl], sem))
    for c in copies:
        c.wait()

    lanes = jnp.arange(16, dtype=jnp.int32)
    gb = gbv[...]

    def group_body(g, _):
        row_v = g * 16 + lanes
        acc = jnp.zeros((16,), dtype=jnp.float32)
        for d in range(EMBED_DIM):
            d_v = jnp.full((16,), d, dtype=jnp.int32)
            pv = plsc.load_gather(prow, [row_v, d_v])
            cv = plsc.load_gather(crow, [row_v, d_v])
            acc = acc + pv * cv
        pbx = plsc.load_gather(pbv, [row_v])
        cbx = plsc.load_gather(cbv, [row_v])
        x = acc + pbx + cbx + gb
        y = 1.0 / (1.0 + jnp.exp(-x))
        outv[pl.ds(pl.multiple_of(g * 16, 16), 16)] = y
        return 0

    lax.fori_loop(0, GROUPS, group_body, 0)

    base = pl.multiple_of(wid * BPW, BPW)
    pltpu.sync_copy(outv, out_hbm.at[pl.ds(base, BPW)])


def kernel(player_ids, course_ids, player_embed, course_embed, player_bias,
           course_bias, global_bias):
    pid = player_ids.astype(jnp.int32).reshape(NW, NCHUNK, CHUNK)
    cid = course_ids.astype(jnp.int32).reshape(NW, NCHUNK, CHUNK)
    pb = player_bias.reshape(N_PLAYERS)
    cb = course_bias.reshape(N_COURSES)
    gb = jnp.broadcast_to(global_bias.astype(jnp.float32), (16,))
    return _sc_kernel(pid, cid, player_embed, course_embed, pb, cb, gb)
